# baseline (speedup 1.0000x reference)
import jax
import jax.numpy as jnp
from jax.experimental import pallas as pl


def _body(x_ref, emb_ref, o_ref, *, n, d):
    e = emb_ref[...]
    for j in range(n):
        o_ref[:, j * d:(j + 1) * d] = x_ref[:, j * d:(j + 1) * d] + e


def kernel(x, emb):
    L, N, D = x.shape
    x2 = x.reshape(L, N * D)
    BL = 256
    grid = (L // BL,)
    import functools
    out = pl.pallas_call(
        functools.partial(_body, n=N, d=D),
        grid=grid,
        in_specs=[
            pl.BlockSpec((BL, N * D), lambda i: (i, 0)),
            pl.BlockSpec((BL, D), lambda i: (i, 0)),
        ],
        out_specs=pl.BlockSpec((BL, N * D), lambda i: (i, 0)),
        out_shape=jax.ShapeDtypeStruct((L, N * D), x.dtype),
    )(x2, emb)
    return out.reshape(L, N, D)


# speedup vs baseline: 3.9689x; 3.9689x over previous
import functools

import jax
import jax.numpy as jnp
from jax.experimental import pallas as pl


def _body(x_ref, emb_ref, o_ref, *, n):
    e = emb_ref[...]
    for j in range(n):
        o_ref[:, j, :] = x_ref[:, j, :] + e


def kernel(x, emb):
    L, N, D = x.shape
    BL = 256
    grid = (L // BL,)
    return pl.pallas_call(
        functools.partial(_body, n=N),
        grid=grid,
        in_specs=[
            pl.BlockSpec((BL, N, D), lambda i: (i, 0, 0)),
            pl.BlockSpec((BL, D), lambda i: (i, 0)),
        ],
        out_specs=pl.BlockSpec((BL, N, D), lambda i: (i, 0, 0)),
        out_shape=jax.ShapeDtypeStruct((L, N, D), x.dtype),
    )(x, emb)


# speedup vs baseline: 4.0028x; 1.0086x over previous
import functools

import jax
import jax.numpy as jnp
from jax.experimental import pallas as pl


def _body(x_ref, emb_ref, o_ref, *, n):
    e = emb_ref[...]
    for j in range(n):
        o_ref[:, j, :] = x_ref[:, j, :] + e


def kernel(x, emb):
    L, N, D = x.shape
    BL = 512
    grid = (L // BL,)
    return pl.pallas_call(
        functools.partial(_body, n=N),
        grid=grid,
        in_specs=[
            pl.BlockSpec((BL, N, D), lambda i: (i, 0, 0)),
            pl.BlockSpec((BL, D), lambda i: (i, 0)),
        ],
        out_specs=pl.BlockSpec((BL, N, D), lambda i: (i, 0, 0)),
        out_shape=jax.ShapeDtypeStruct((L, N, D), x.dtype),
    )(x, emb)
